# baseline (speedup 1.0000x reference)
import functools

import jax
import jax.numpy as jnp
from jax import lax
from jax.experimental import pallas as pl
from jax.experimental.pallas import tpu as pltpu
from jax.experimental.pallas import tpu_sc as plsc

N = 10000
E = 160000
D = 256
NT_N = 10
TN = N // NT_N
NT_E = 160
TE = E // NT_E

_NEG_INF = float("-inf")


def _enc_body(nodes_ref, coords_ref, w1n, b1n, w2n, b2n, w1c, b1c, w2c, b2c,
              h_ref, hc_ref):
    t = jnp.maximum(
        jnp.dot(nodes_ref[...], w1n[...], preferred_element_type=jnp.float32)
        + b1n[...], 0.0)
    h_ref[...] = jnp.dot(t, w2n[...], preferred_element_type=jnp.float32) + b2n[...]
    t2 = jnp.maximum(
        jnp.dot(coords_ref[...], w1c[...], preferred_element_type=jnp.float32)
        + b1c[...], 0.0)
    hc_ref[...] = jnp.dot(t2, w2c[...], preferred_element_type=jnp.float32) + b2c[...]


def _encode(nodes, coords8, hn_W1, hn_b1, hn_W2, hn_b2, hc_W1p, hc_b1, hc_W2, hc_b2):
    full = lambda shape: pl.BlockSpec(shape, lambda i: (0, 0))
    return pl.pallas_call(
        _enc_body,
        grid=(NT_N,),
        in_specs=[
            pl.BlockSpec((TN, 128), lambda i: (i, 0)),
            pl.BlockSpec((TN, 8), lambda i: (i, 0)),
            full((128, D)), full((1, D)), full((D, D)), full((1, D)),
            full((8, D)), full((1, D)), full((D, D)), full((1, D)),
        ],
        out_specs=[
            pl.BlockSpec((TN, D), lambda i: (i, 0)),
            pl.BlockSpec((TN, D), lambda i: (i, 0)),
        ],
        out_shape=[
            jax.ShapeDtypeStruct((N, D), jnp.float32),
            jax.ShapeDtypeStruct((N, D), jnp.float32),
        ],
    )(nodes, coords8, hn_W1, hn_b1.reshape(1, D), hn_W2, hn_b2.reshape(1, D),
      hc_W1p, hc_b1.reshape(1, D), hc_W2, hc_b2.reshape(1, D))


def _st_body(h_ref, hc_ref, wa, wb, wc, b1, s_ref, t_ref):
    h = h_ref[...]
    hcwc = jnp.dot(hc_ref[...], wc[...], preferred_element_type=jnp.float32)
    s_ref[...] = (jnp.dot(h, wa[...], preferred_element_type=jnp.float32)
                  + hcwc + b1[...])
    t_ref[...] = (jnp.dot(h, wb[...], preferred_element_type=jnp.float32)
                  - hcwc)


def _node_transform(h, hc, wa, wb, wc, b1):
    full = lambda: pl.BlockSpec((D, D), lambda i: (0, 0))
    return pl.pallas_call(
        _st_body,
        grid=(NT_N,),
        in_specs=[
            pl.BlockSpec((TN, D), lambda i: (i, 0)),
            pl.BlockSpec((TN, D), lambda i: (i, 0)),
            full(), full(), full(),
            pl.BlockSpec((1, D), lambda i: (0, 0)),
        ],
        out_specs=[
            pl.BlockSpec((TN, D), lambda i: (i, 0)),
            pl.BlockSpec((TN, D), lambda i: (i, 0)),
        ],
        out_shape=[
            jax.ShapeDtypeStruct((N, D), jnp.float32),
            jax.ShapeDtypeStruct((N, D), jnp.float32),
        ],
    )(h, hc, wa, wb, wc, b1.reshape(1, D))


_CH = 200


def _gather2(S, T, src, dst):
    info = plsc.get_sparse_core_info()
    nc, ns = info.num_cores, info.num_subcores
    nw = nc * ns
    epw = E // nw
    nch = epw // _CH
    mesh = plsc.VectorSubcoreMesh(core_axis_name="c", subcore_axis_name="s")

    @functools.partial(
        pl.kernel,
        out_type=(jax.ShapeDtypeStruct((E, D), jnp.float32),
                  jax.ShapeDtypeStruct((E, D), jnp.float32)),
        mesh=mesh,
        scratch_types=[
            pltpu.VMEM((_CH,), jnp.int32),
            pltpu.VMEM((_CH,), jnp.int32),
            pltpu.VMEM((_CH, D), jnp.float32),
            pltpu.VMEM((_CH, D), jnp.float32),
            pltpu.SemaphoreType.DMA,
            pltpu.SemaphoreType.DMA,
        ],
    )
    def k(S_hbm, T_hbm, src_hbm, dst_hbm, P_hbm, Q_hbm,
          si_v, di_v, sr_v, dr_v, sem1, sem2):
        wid = lax.axis_index("s") * nc + lax.axis_index("c")
        base_w = wid * epw
        for c in range(nch):
            base = base_w + c * _CH
            pltpu.sync_copy(src_hbm.at[pl.ds(base, _CH)], si_v)
            pltpu.sync_copy(dst_hbm.at[pl.ds(base, _CH)], di_v)
            cp1 = pltpu.async_copy(S_hbm.at[si_v], sr_v, sem1)
            cp2 = pltpu.async_copy(T_hbm.at[di_v], dr_v, sem2)
            cp1.wait()
            cp2.wait()
            pltpu.sync_copy(sr_v, P_hbm.at[pl.ds(base, _CH)])
            pltpu.sync_copy(dr_v, Q_hbm.at[pl.ds(base, _CH)])

    return k(S, T, src, dst)


def _edge_body(p_ref, q_ref, w2, b2, m_ref):
    a = jnp.maximum(p_ref[...] + q_ref[...], 0.0)
    m_ref[...] = jnp.dot(a, w2[...], preferred_element_type=jnp.float32) + b2[...]


def _edge_mlp(P, Q, w2, b2):
    return pl.pallas_call(
        _edge_body,
        grid=(NT_E,),
        in_specs=[
            pl.BlockSpec((TE, D), lambda i: (i, 0)),
            pl.BlockSpec((TE, D), lambda i: (i, 0)),
            pl.BlockSpec((D, D), lambda i: (0, 0)),
            pl.BlockSpec((1, D), lambda i: (0, 0)),
        ],
        out_specs=pl.BlockSpec((TE, D), lambda i: (i, 0)),
        out_shape=jax.ShapeDtypeStruct((E, D), jnp.float32),
    )(P, Q, w2, b2.reshape(1, D))


_WR = 313
_SCH = 2000
_CB = 64


def _scatter_max(M, dst):
    info = plsc.get_sparse_core_info()
    nc, ns = info.num_cores, info.num_subcores
    nw = nc * ns
    nchunks = E // _SCH
    mesh = plsc.VectorSubcoreMesh(core_axis_name="c", subcore_axis_name="s")

    @functools.partial(
        pl.kernel,
        out_type=jax.ShapeDtypeStruct((nw * _WR * D,), jnp.float32),
        mesh=mesh,
        compiler_params=pltpu.CompilerParams(needs_layout_passes=False),
        scratch_types=[
            pltpu.VMEM((_WR * D,), jnp.float32),
            pltpu.VMEM((_SCH,), jnp.int32),
            pltpu.VMEM((_SCH + _CB,), jnp.int32),
            pltpu.VMEM((_SCH + _CB,), jnp.int32),
            pltpu.VMEM((_CB, D), jnp.float32),
            pltpu.VMEM((_CB,), jnp.int32),
            pltpu.SemaphoreType.DMA,
        ],
    )
    def k(M_hbm, dst_hbm, agg_hbm, acc, dstv, mid, mld, rows, midb, sem):
        wid = lax.axis_index("s") * nc + lax.axis_index("c")
        lo = wid * _WR
        neg = jnp.full((16,), _NEG_INF, jnp.float32)
        iota = lax.iota(jnp.int32, 16)

        def initacc(i, _):
            acc[pl.ds(i * 16, 16)] = neg
            return 0
        lax.fori_loop(0, _WR * D // 16, initacc, 0)

        def initmid(i, _):
            mid[pl.ds(i * 16, 16)] = jnp.zeros((16,), jnp.int32)
            mld[pl.ds(i * 16, 16)] = jnp.zeros((16,), jnp.int32)
            return 0
        lax.fori_loop(0, (_SCH + _CB) // 16, initmid, 0)

        def chunk(c, _):
            pltpu.sync_copy(dst_hbm.at[pl.ds(c * _SCH, _SCH)], dstv)

            def scan_g(g, cnt):
                ld = dstv[pl.ds(g * 16, 16)] - lo
                msk = (ld >= 0) & (ld < _WR)
                eid = c * _SCH + g * 16 + iota
                inc = plsc.cumsum(msk.astype(jnp.int32))
                pos = cnt + inc - 1
                plsc.store_scatter(mid, [pos], eid, mask=msk)
                plsc.store_scatter(mld, [pos], ld, mask=msk)
                return cnt + lax.reduce_max(inc, (0,))

            cnt = lax.fori_loop(0, _SCH // 16, scan_g, 0)
            nbat = (cnt + _CB - 1) // _CB

            def batch(b, _):
                for i in range(_CB // 16):
                    midb[pl.ds(i * 16, 16)] = mid[pl.ds(b * _CB + i * 16, 16)]
                pltpu.async_copy(M_hbm.at[midb], rows, sem).wait()

                def row(r, _):
                    gr = b * _CB + r
                    lds = plsc.load_gather(mld, [jnp.full((16,), gr, jnp.int32)])
                    lds = jnp.clip(lds, 0, _WR - 1)
                    selv = jnp.full((16,), gr, jnp.int32) < jnp.full((16,), cnt, jnp.int32)
                    for j in range(D // 16):
                        addr = lds * D + iota + j * 16
                        a = plsc.load_gather(acc, [addr])
                        mrow = plsc.load_gather(
                            rows, [jnp.full((16,), r, jnp.int32), iota + j * 16])
                        mrow = jnp.where(selv, mrow, neg)
                        plsc.store_scatter(acc, [addr], jnp.maximum(a, mrow))
                    return 0

                lax.fori_loop(0, _CB, row, 0)
                return 0

            lax.fori_loop(0, nbat, batch, 0)
            return 0

        lax.fori_loop(0, nchunks, chunk, 0)
        pltpu.sync_copy(acc, agg_hbm.at[pl.ds(lo * D, _WR * D)])

    return k(M, dst)


def _upd_body(h_ref, agg_ref, out_ref):
    agg = agg_ref[...]
    out_ref[...] = h_ref[...] + jnp.where(jnp.isneginf(agg), 0.0, agg)


def _h_update(h, agg):
    return pl.pallas_call(
        _upd_body,
        grid=(NT_N,),
        in_specs=[
            pl.BlockSpec((TN, D), lambda i: (i, 0)),
            pl.BlockSpec((TN, D), lambda i: (i, 0)),
        ],
        out_specs=pl.BlockSpec((TN, D), lambda i: (i, 0)),
        out_shape=jax.ShapeDtypeStruct((N, D), jnp.float32),
    )(h, agg)


def kernel(nodes, coords, edge_index, hn_W1, hn_b1, hn_W2, hn_b2,
           hc_W1, hc_b1, hc_W2, hc_b2, mp_W1, mp_b1, mp_W2, mp_b2):
    src = edge_index[0]
    dst = edge_index[1]
    coords8 = jnp.pad(coords, ((0, 0), (0, 5)))
    hc_W1p = jnp.pad(hc_W1, ((0, 5), (0, 0)))

    h, hcv = _encode(nodes, coords8, hn_W1, hn_b1, hn_W2, hn_b2,
                     hc_W1p, hc_b1, hc_W2, hc_b2)

    for i in range(3):
        wa = mp_W1[i, 0:D, :]
        wb = mp_W1[i, D:2 * D, :]
        wc = mp_W1[i, 2 * D:3 * D, :]
        S, T = _node_transform(h, hcv, wa, wb, wc, mp_b1[i])
        P, Q = _gather2(S, T, src, dst)
        M = _edge_mlp(P, Q, mp_W2[i], mp_b2[i])
        agg = _scatter_max(M, dst).reshape(-1, D)[:N]
        h = _h_update(h, agg)
    return h


# speedup vs baseline: 2.4821x; 2.4821x over previous
import functools

import jax
import jax.numpy as jnp
from jax import lax
from jax.experimental import pallas as pl
from jax.experimental.pallas import tpu as pltpu
from jax.experimental.pallas import tpu_sc as plsc

N = 10000
E = 160000
D = 256
NT_N = 10
TN = N // NT_N
NT_E = 160
TE = E // NT_E

_NEG_INF = float("-inf")


def _enc_body(nodes_ref, coords_ref, w1n, b1n, w2n, b2n, w1c, b1c, w2c, b2c,
              h_ref, hc_ref):
    t = jnp.maximum(
        jnp.dot(nodes_ref[...], w1n[...], preferred_element_type=jnp.float32)
        + b1n[...], 0.0)
    h_ref[...] = jnp.dot(t, w2n[...], preferred_element_type=jnp.float32) + b2n[...]
    t2 = jnp.maximum(
        jnp.dot(coords_ref[...], w1c[...], preferred_element_type=jnp.float32)
        + b1c[...], 0.0)
    hc_ref[...] = jnp.dot(t2, w2c[...], preferred_element_type=jnp.float32) + b2c[...]


def _encode(nodes, coords8, hn_W1, hn_b1, hn_W2, hn_b2, hc_W1p, hc_b1, hc_W2, hc_b2):
    full = lambda shape: pl.BlockSpec(shape, lambda i: (0, 0))
    return pl.pallas_call(
        _enc_body,
        grid=(NT_N,),
        in_specs=[
            pl.BlockSpec((TN, 128), lambda i: (i, 0)),
            pl.BlockSpec((TN, 8), lambda i: (i, 0)),
            full((128, D)), full((1, D)), full((D, D)), full((1, D)),
            full((8, D)), full((1, D)), full((D, D)), full((1, D)),
        ],
        out_specs=[
            pl.BlockSpec((TN, D), lambda i: (i, 0)),
            pl.BlockSpec((TN, D), lambda i: (i, 0)),
        ],
        out_shape=[
            jax.ShapeDtypeStruct((N, D), jnp.float32),
            jax.ShapeDtypeStruct((N, D), jnp.float32),
        ],
    )(nodes, coords8, hn_W1, hn_b1.reshape(1, D), hn_W2, hn_b2.reshape(1, D),
      hc_W1p, hc_b1.reshape(1, D), hc_W2, hc_b2.reshape(1, D))


def _st_body(h_ref, hc_ref, wa, wb, wc, b1, s_ref, t_ref):
    h = h_ref[...]
    hcwc = jnp.dot(hc_ref[...], wc[...], preferred_element_type=jnp.float32)
    s_ref[...] = (jnp.dot(h, wa[...], preferred_element_type=jnp.float32)
                  + hcwc + b1[...])
    t_ref[...] = (jnp.dot(h, wb[...], preferred_element_type=jnp.float32)
                  - hcwc)


def _node_transform(h, hc, wa, wb, wc, b1):
    full = lambda: pl.BlockSpec((D, D), lambda i: (0, 0))
    return pl.pallas_call(
        _st_body,
        grid=(NT_N,),
        in_specs=[
            pl.BlockSpec((TN, D), lambda i: (i, 0)),
            pl.BlockSpec((TN, D), lambda i: (i, 0)),
            full(), full(), full(),
            pl.BlockSpec((1, D), lambda i: (0, 0)),
        ],
        out_specs=[
            pl.BlockSpec((TN, D), lambda i: (i, 0)),
            pl.BlockSpec((TN, D), lambda i: (i, 0)),
        ],
        out_shape=[
            jax.ShapeDtypeStruct((N, D), jnp.float32),
            jax.ShapeDtypeStruct((N, D), jnp.float32),
        ],
    )(h, hc, wa, wb, wc, b1.reshape(1, D))


_CH = 200


def _gather2(S, T, src, dst):
    info = plsc.get_sparse_core_info()
    nc, ns = info.num_cores, info.num_subcores
    nw = nc * ns
    epw = E // nw
    nch = epw // _CH
    mesh = plsc.VectorSubcoreMesh(core_axis_name="c", subcore_axis_name="s")

    @functools.partial(
        pl.kernel,
        out_type=(jax.ShapeDtypeStruct((E, D), jnp.float32),
                  jax.ShapeDtypeStruct((E, D), jnp.float32)),
        mesh=mesh,
        scratch_types=[
            pltpu.VMEM((_CH,), jnp.int32),
            pltpu.VMEM((_CH,), jnp.int32),
            pltpu.VMEM((_CH, D), jnp.float32),
            pltpu.VMEM((_CH, D), jnp.float32),
            pltpu.SemaphoreType.DMA,
            pltpu.SemaphoreType.DMA,
        ],
    )
    def k(S_hbm, T_hbm, src_hbm, dst_hbm, P_hbm, Q_hbm,
          si_v, di_v, sr_v, dr_v, sem1, sem2):
        wid = lax.axis_index("s") * nc + lax.axis_index("c")
        base_w = wid * epw
        for c in range(nch):
            base = base_w + c * _CH
            pltpu.sync_copy(src_hbm.at[pl.ds(base, _CH)], si_v)
            pltpu.sync_copy(dst_hbm.at[pl.ds(base, _CH)], di_v)
            cp1 = pltpu.async_copy(S_hbm.at[si_v], sr_v, sem1)
            cp2 = pltpu.async_copy(T_hbm.at[di_v], dr_v, sem2)
            cp1.wait()
            cp2.wait()
            pltpu.sync_copy(sr_v, P_hbm.at[pl.ds(base, _CH)])
            pltpu.sync_copy(dr_v, Q_hbm.at[pl.ds(base, _CH)])

    return k(S, T, src, dst)


def _edge_body(p_ref, q_ref, w2, b2, m_ref):
    a = jnp.maximum(p_ref[...] + q_ref[...], 0.0)
    m_ref[...] = jnp.dot(a, w2[...], preferred_element_type=jnp.float32) + b2[...]


def _edge_mlp(P, Q, w2, b2):
    return pl.pallas_call(
        _edge_body,
        grid=(NT_E,),
        in_specs=[
            pl.BlockSpec((TE, D), lambda i: (i, 0)),
            pl.BlockSpec((TE, D), lambda i: (i, 0)),
            pl.BlockSpec((D, D), lambda i: (0, 0)),
            pl.BlockSpec((1, D), lambda i: (0, 0)),
        ],
        out_specs=pl.BlockSpec((TE, D), lambda i: (i, 0)),
        out_shape=jax.ShapeDtypeStruct((E, D), jnp.float32),
    )(P, Q, w2, b2.reshape(1, D))


_WR = 313
_SCH = 2000
_CB = 64
_RING = 4096
_FL = 2048
_PLN = E + 2 * _FL
_PAD_PK = _WR


def _sc_mesh_info():
    info = plsc.get_sparse_core_info()
    return info.num_cores, info.num_subcores


def _scatter_plan(dst):
    nc, ns = _sc_mesh_info()
    nw = nc * ns
    nchunks = E // _SCH
    mesh = plsc.VectorSubcoreMesh(core_axis_name="c", subcore_axis_name="s")

    @functools.partial(
        pl.kernel,
        out_type=(jax.ShapeDtypeStruct((nw * _PLN,), jnp.int32),
                  jax.ShapeDtypeStruct((nw * 8,), jnp.int32)),
        mesh=mesh,
        compiler_params=pltpu.CompilerParams(needs_layout_passes=False),
        scratch_types=[
            pltpu.VMEM((_RING,), jnp.int32),
            pltpu.VMEM((_SCH,), jnp.int32),
            pltpu.VMEM((16,), jnp.int32),
        ],
    )
    def k(dst_hbm, plan_hbm, cnt_hbm, ring, dstv, cbuf):
        wid = lax.axis_index("s") * nc + lax.axis_index("c")
        lo = wid * _WR
        base_out = wid * _PLN
        iota = lax.iota(jnp.int32, 16)

        def flush(flushed):
            fl8 = pl.multiple_of(flushed, _FL)
            half = (fl8 >> 11) & 1
            pltpu.sync_copy(ring.at[pl.ds(half * _FL, _FL)],
                            plan_hbm.at[pl.ds(base_out + fl8, _FL)])
            return flushed + _FL

        def chunk(c, carry):
            cntv, flushed = carry
            pltpu.sync_copy(dst_hbm.at[pl.ds(c * _SCH, _SCH)], dstv)

            def scan_g(g, cntv):
                ld = dstv[pl.ds(g * 16, 16)] - lo
                msk = (ld >= 0) & (ld < _WR)
                pk = ((c * _SCH + g * 16 + iota) << 9) | (ld & 511)
                pos = (cntv + plsc.cumsum(msk.astype(jnp.int32)) - 1) & (_RING - 1)
                plsc.store_scatter(ring, [pos], pk, mask=msk)
                return cntv + plsc.all_reduce_population_count(msk)

            cntv = lax.fori_loop(0, _SCH // 16, scan_g, cntv)
            cnt_s = lax.reduce_max(cntv, (0,))

            def maybe_flush(flushed):
                return lax.cond(cnt_s - flushed >= _FL, flush,
                                lambda f: f, flushed)

            return cntv, maybe_flush(flushed)

        cntv, flushed = lax.fori_loop(
            0, nchunks, chunk,
            (jnp.zeros((16,), jnp.int32), jnp.int32(0)))
        cnt_s = lax.reduce_max(cntv, (0,))
        padv = jnp.full((16,), _PAD_PK, jnp.int32)
        for kk in range(4):
            plsc.store_scatter(ring, [(cnt_s + kk * 16 + iota) & (_RING - 1)], padv)

        def cond(fl):
            return fl < cnt_s + _CB

        flushed = lax.while_loop(cond, flush, flushed)
        cbuf[pl.ds(0, 16)] = cntv
        pltpu.sync_copy(cbuf.at[pl.ds(0, 8)], cnt_hbm.at[pl.ds(wid * 8, 8)])

    return k(dst)


def _scatter_max2(M, plan, counts):
    nc, ns = _sc_mesh_info()
    nw = nc * ns
    mesh = plsc.VectorSubcoreMesh(core_axis_name="c", subcore_axis_name="s")

    @functools.partial(
        pl.kernel,
        out_type=jax.ShapeDtypeStruct((nw * _WR * D,), jnp.float32),
        mesh=mesh,
        compiler_params=pltpu.CompilerParams(needs_layout_passes=False),
        scratch_types=[
            pltpu.VMEM(((_WR + 1) * D,), jnp.float32),
            pltpu.VMEM((_CB,), jnp.int32),
            pltpu.VMEM((_CB,), jnp.int32),
            pltpu.VMEM((_CB,), jnp.int32),
            pltpu.VMEM((_CB,), jnp.int32),
            pltpu.VMEM((_CB,), jnp.int32),
            pltpu.VMEM((_CB,), jnp.int32),
            pltpu.VMEM((_CB, D), jnp.float32),
            pltpu.VMEM((_CB, D), jnp.float32),
            pltpu.VMEM((16,), jnp.int32),
            pltpu.SemaphoreType.DMA,
            pltpu.SemaphoreType.DMA,
        ],
    )
    def k(M_hbm, plan_hbm, cnt_hbm, agg_hbm, acc,
          pke0, pke1, mid0, mid1, mld0, mld1, rows0, rows1, cbuf, sem0, sem1):
        wid = lax.axis_index("s") * nc + lax.axis_index("c")
        lo = wid * _WR
        base_in = wid * _PLN
        iota = lax.iota(jnp.int32, 16)
        neg = jnp.full((16,), _NEG_INF, jnp.float32)

        def initacc(i, _):
            acc[pl.ds(i * 16, 16)] = neg
            return 0
        lax.fori_loop(0, (_WR + 1) * D // 16, initacc, 0)

        cbuf[pl.ds(0, 16)] = jnp.zeros((16,), jnp.int32)
        pltpu.sync_copy(cnt_hbm.at[pl.ds(wid * 8, 8)], cbuf.at[pl.ds(0, 8)])
        cnt = lax.reduce_max(cbuf[pl.ds(0, 16)], (0,))
        nbat = (cnt + _CB - 1) // _CB

        slots = ((pke0, mid0, mld0, rows0, sem0),
                 (pke1, mid1, mld1, rows1, sem1))

        def stage(b, slot):
            pke, mid, mld, rows, sem = slots[slot]
            pltpu.sync_copy(plan_hbm.at[pl.ds(base_in + b * _CB, _CB)], pke)
            for i in range(_CB // 16):
                pk = pke[pl.ds(i * 16, 16)]
                mid[pl.ds(i * 16, 16)] = pk >> 9
                mld[pl.ds(i * 16, 16)] = pk & 511
            return pltpu.async_copy(M_hbm.at[mid], rows, sem)

        def process(slot):
            _, _, mld, rows, _ = slots[slot]

            def row(r, _):
                lds = plsc.load_gather(mld, [jnp.full((16,), r, jnp.int32)])
                lds = jnp.minimum(lds, _WR)
                for j in range(D // 16):
                    addr = lds * D + iota + j * 16
                    a = plsc.load_gather(acc, [addr])
                    mrow = plsc.load_gather(
                        rows, [jnp.full((16,), r, jnp.int32), iota + j * 16])
                    plsc.store_scatter(acc, [addr], jnp.maximum(a, mrow))
                return 0

            lax.fori_loop(0, _CB, row, 0)

        @pl.when(nbat > 0)
        def _run():
            stage(0, 0)

            def pair(i, _):
                b1 = 2 * i + 1

                @pl.when(b1 < nbat)
                def _s1():
                    stage(b1, 1)

                pltpu.make_async_copy(M_hbm.at[slots[0][1]], slots[0][3],
                                      slots[0][4]).wait()
                process(0)

                @pl.when(b1 + 1 < nbat)
                def _s0():
                    stage(b1 + 1, 0)

                @pl.when(b1 < nbat)
                def _p1():
                    pltpu.make_async_copy(M_hbm.at[slots[1][1]], slots[1][3],
                                          slots[1][4]).wait()
                    process(1)
                return 0

            lax.fori_loop(0, (nbat + 1) // 2, pair, 0)

        pltpu.sync_copy(acc.at[pl.ds(0, _WR * D)],
                        agg_hbm.at[pl.ds(lo * D, _WR * D)])

    return k(M, plan, counts)


def _upd_body(h_ref, agg_ref, out_ref):
    agg = agg_ref[...]
    out_ref[...] = h_ref[...] + jnp.where(jnp.isneginf(agg), 0.0, agg)


def _h_update(h, agg):
    return pl.pallas_call(
        _upd_body,
        grid=(NT_N,),
        in_specs=[
            pl.BlockSpec((TN, D), lambda i: (i, 0)),
            pl.BlockSpec((TN, D), lambda i: (i, 0)),
        ],
        out_specs=pl.BlockSpec((TN, D), lambda i: (i, 0)),
        out_shape=jax.ShapeDtypeStruct((N, D), jnp.float32),
    )(h, agg)


def kernel(nodes, coords, edge_index, hn_W1, hn_b1, hn_W2, hn_b2,
           hc_W1, hc_b1, hc_W2, hc_b2, mp_W1, mp_b1, mp_W2, mp_b2):
    src = edge_index[0]
    dst = edge_index[1]
    coords8 = jnp.pad(coords, ((0, 0), (0, 5)))
    hc_W1p = jnp.pad(hc_W1, ((0, 5), (0, 0)))

    h, hcv = _encode(nodes, coords8, hn_W1, hn_b1, hn_W2, hn_b2,
                     hc_W1p, hc_b1, hc_W2, hc_b2)

    plan, counts = _scatter_plan(dst)

    for i in range(3):
        wa = mp_W1[i, 0:D, :]
        wb = mp_W1[i, D:2 * D, :]
        wc = mp_W1[i, 2 * D:3 * D, :]
        S, T = _node_transform(h, hcv, wa, wb, wc, mp_b1[i])
        P, Q = _gather2(S, T, src, dst)
        M = _edge_mlp(P, Q, mp_W2[i], mp_b2[i])
        agg = _scatter_max2(M, plan, counts).reshape(-1, D)[:N]
        h = _h_update(h, agg)
    return h
